# baseline (speedup 1.0000x reference)
import functools

import jax
import jax.numpy as jnp
from jax import lax
from jax.experimental import pallas as pl
from jax.experimental.pallas import tpu as pltpu
from jax.experimental.pallas import tpu_sc as plsc

N_NODES = 10000
N_EDGES = 160000
IN_DIM = 256
HID_DIM = 128
OUT_DIM = 64

NC, NS = 2, 16
NW = NC * NS
N_PAD = 10240
ROWS_PER_TILE = N_PAD // NS
EB = 128
BLK_PER_W = -(-N_EDGES // (EB * NW))
E_PAD = BLK_PER_W * EB * NW

_TC_ROWS = 1024
_GRID = N_PAD // _TC_ROWS


def _dense1_body(x_ref, wl_ref, wr_ref, b_ref, p_ref, q_ref):
    x = x_ref[...]
    dn = (((1,), (1,)), ((), ()))
    p_ref[...] = lax.dot_general(x, wl_ref[...], dn,
                                 preferred_element_type=jnp.float32)
    q_ref[...] = lax.dot_general(x, wr_ref[...], dn,
                                 preferred_element_type=jnp.float32) + b_ref[...]


@jax.jit
def _dense1(xp, w1l, w1r, b1):
    return pl.pallas_call(
        _dense1_body,
        grid=(_GRID,),
        in_specs=[
            pl.BlockSpec((_TC_ROWS, IN_DIM), lambda i: (i, 0)),
            pl.BlockSpec((HID_DIM, IN_DIM), lambda i: (0, 0)),
            pl.BlockSpec((HID_DIM, IN_DIM), lambda i: (0, 0)),
            pl.BlockSpec((1, HID_DIM), lambda i: (0, 0)),
        ],
        out_specs=[
            pl.BlockSpec((_TC_ROWS, HID_DIM), lambda i: (i, 0)),
            pl.BlockSpec((_TC_ROWS, HID_DIM), lambda i: (i, 0)),
        ],
        out_shape=[
            jax.ShapeDtypeStruct((N_PAD, HID_DIM), jnp.float32),
            jax.ShapeDtypeStruct((N_PAD, HID_DIM), jnp.float32),
        ],
    )(xp, w1l, w1r, b1)


def _mid_body(s0_ref, s1_ref, d0_ref, d1_ref, q1_ref, wl_ref, wr_ref, b_ref,
              p_ref, q_ref, invd_ref):
    deg = jnp.maximum(d0_ref[...] + d1_ref[...], 1.0)
    h = jnp.maximum((s0_ref[...] + s1_ref[...]) / deg + q1_ref[...], 0.0)
    dn = (((1,), (1,)), ((), ()))
    p_ref[...] = lax.dot_general(h, wl_ref[...], dn,
                                 preferred_element_type=jnp.float32)
    q_ref[...] = lax.dot_general(h, wr_ref[...], dn,
                                 preferred_element_type=jnp.float32) + b_ref[...]
    invd_ref[...] = 1.0 / deg


@jax.jit
def _mid(s0, s1, d0, d1, q1, w2l, w2r, b2):
    return pl.pallas_call(
        _mid_body,
        grid=(_GRID,),
        in_specs=[
            pl.BlockSpec((_TC_ROWS, HID_DIM), lambda i: (i, 0)),
            pl.BlockSpec((_TC_ROWS, HID_DIM), lambda i: (i, 0)),
            pl.BlockSpec((_TC_ROWS, 1), lambda i: (i, 0)),
            pl.BlockSpec((_TC_ROWS, 1), lambda i: (i, 0)),
            pl.BlockSpec((_TC_ROWS, HID_DIM), lambda i: (i, 0)),
            pl.BlockSpec((OUT_DIM, HID_DIM), lambda i: (0, 0)),
            pl.BlockSpec((OUT_DIM, HID_DIM), lambda i: (0, 0)),
            pl.BlockSpec((1, OUT_DIM), lambda i: (0, 0)),
        ],
        out_specs=[
            pl.BlockSpec((_TC_ROWS, OUT_DIM), lambda i: (i, 0)),
            pl.BlockSpec((_TC_ROWS, OUT_DIM), lambda i: (i, 0)),
            pl.BlockSpec((_TC_ROWS, 1), lambda i: (i, 0)),
        ],
        out_shape=[
            jax.ShapeDtypeStruct((N_PAD, OUT_DIM), jnp.float32),
            jax.ShapeDtypeStruct((N_PAD, OUT_DIM), jnp.float32),
            jax.ShapeDtypeStruct((N_PAD, 1), jnp.float32),
        ],
    )(s0, s1, d0, d1, q1, w2l, w2r, b2)


def _final_body(t0_ref, t1_ref, invd_ref, q2_ref, o_ref):
    o_ref[...] = jnp.maximum(
        (t0_ref[...] + t1_ref[...]) * invd_ref[...] + q2_ref[...], 0.0)


@jax.jit
def _final(t0, t1, invd, q2):
    return pl.pallas_call(
        _final_body,
        grid=(_GRID,),
        in_specs=[
            pl.BlockSpec((_TC_ROWS, OUT_DIM), lambda i: (i, 0)),
            pl.BlockSpec((_TC_ROWS, OUT_DIM), lambda i: (i, 0)),
            pl.BlockSpec((_TC_ROWS, 1), lambda i: (i, 0)),
            pl.BlockSpec((_TC_ROWS, OUT_DIM), lambda i: (i, 0)),
        ],
        out_specs=[pl.BlockSpec((_TC_ROWS, OUT_DIM), lambda i: (i, 0))],
        out_shape=[jax.ShapeDtypeStruct((N_PAD, OUT_DIM), jnp.float32)],
    )(t0, t1, invd, q2)


@functools.cache
def _make_sc_scatter(d, with_deg):
    mesh = plsc.VectorSubcoreMesh(core_axis_name="c", subcore_axis_name="s",
                                  num_cores=NC, num_subcores=NS)
    out_type = [jax.ShapeDtypeStruct((N_PAD, d), jnp.float32)] * 2
    if with_deg:
        out_type += [jax.ShapeDtypeStruct((N_PAD,), jnp.float32)] * 2
    scratch = [
        pltpu.VMEM((BLK_PER_W, EB), jnp.int32),
        pltpu.VMEM((BLK_PER_W, EB), jnp.int32),
        pltpu.VMEM((EB, d), jnp.float32),
        pltpu.VMEM((EB, d), jnp.float32),
        pltpu.VMEM_SHARED((N_PAD, d), jnp.float32),
        pltpu.SemaphoreType.DMA,
        pltpu.SemaphoreType.DMA,
        pltpu.SemaphoreType.DMA,
        pltpu.SemaphoreType.DMA,
    ]
    if with_deg:
        scratch += [
            pltpu.VMEM((EB,), jnp.float32),
            pltpu.VMEM((128,), jnp.float32),
            pltpu.VMEM_SHARED((N_PAD,), jnp.float32),
            pltpu.SemaphoreType.DMA,
        ]

    def body(p_hbm, src_hbm, dst_hbm, *rest):
        if with_deg:
            (out0, out1, dg0, dg1, srcv, dstv, rows0, rows1, acc,
             gs0, gs1, ss0, ss1, onesv, zv, dacc, dsem) = rest
        else:
            (out0, out1, srcv, dstv, rows0, rows1, acc,
             gs0, gs1, ss0, ss1) = rest
            dg0 = dg1 = onesv = zv = dacc = dsem = None
        cid = lax.axis_index("c")
        sid = lax.axis_index("s")
        wid = sid * NC + cid
        base = sid * ROWS_PER_TILE
        rows = (rows0, rows1)
        gsem = (gs0, gs1)
        ssem = (ss0, ss1)

        pltpu.sync_copy(src_hbm.at[pl.ds(wid * BLK_PER_W, BLK_PER_W)], srcv)
        pltpu.sync_copy(dst_hbm.at[pl.ds(wid * BLK_PER_W, BLK_PER_W)], dstv)

        nvec = d // 16

        def zrow(i, _):
            r = i // nvec
            c = (i % nvec) * 16
            rows0[r, pl.ds(c, 16)] = jnp.zeros((16,), jnp.float32)
            return 0

        lax.fori_loop(0, EB * nvec, zrow, 0)
        for k in range(ROWS_PER_TILE // EB):
            pltpu.sync_copy(rows0, acc.at[pl.ds(base + k * EB, EB)])
        if with_deg:
            def fill(i, _):
                onesv[pl.ds(i * 16, 16)] = jnp.ones((16,), jnp.float32)
                zv[pl.ds(i * 16, 16)] = jnp.zeros((16,), jnp.float32)
                return 0

            lax.fori_loop(0, EB // 16, fill, 0)
            for k in range(ROWS_PER_TILE // 128):
                pltpu.sync_copy(zv, dacc.at[pl.ds(base + k * 128, 128)])
        plsc.subcore_barrier()

        def step(j, _):
            pltpu.async_copy(p_hbm.at[srcv.at[j]], rows[0], gsem[0]).wait()
            pltpu.sync_copy(rows[0], acc.at[dstv.at[j]], add=True)
            if with_deg:
                pltpu.sync_copy(onesv, dacc.at[dstv.at[j]], add=True)
            return 0

        lax.fori_loop(0, BLK_PER_W, step, 0)
        plsc.subcore_barrier()

        @pl.when(cid == 0)
        def _():
            pltpu.sync_copy(acc.at[pl.ds(base, ROWS_PER_TILE)],
                            out0.at[pl.ds(base, ROWS_PER_TILE)])
            if with_deg:
                pltpu.sync_copy(dacc.at[pl.ds(base, ROWS_PER_TILE)],
                                dg0.at[pl.ds(base, ROWS_PER_TILE)])

        @pl.when(cid == 1)
        def _():
            pltpu.sync_copy(acc.at[pl.ds(base, ROWS_PER_TILE)],
                            out1.at[pl.ds(base, ROWS_PER_TILE)])
            if with_deg:
                pltpu.sync_copy(dacc.at[pl.ds(base, ROWS_PER_TILE)],
                                dg1.at[pl.ds(base, ROWS_PER_TILE)])

    return pl.kernel(body, out_type=out_type, mesh=mesh,
                     scratch_types=scratch,
                     compiler_params=pltpu.CompilerParams(
                         use_tc_tiling_on_sc=False))


def kernel(x, edge_index, W1_l, b1, W1_r, W2_l, b2, W2_r):
    npad_e = E_PAD - N_EDGES
    src = jnp.concatenate([edge_index[0].astype(jnp.int32),
                           jnp.zeros((npad_e,), jnp.int32)])
    dst = jnp.concatenate([edge_index[1].astype(jnp.int32),
                           N_NODES + (jnp.arange(npad_e, dtype=jnp.int32)
                                      % (N_PAD - N_NODES))])
    src = src.reshape(E_PAD // EB, EB)
    dst = dst.reshape(E_PAD // EB, EB)
    xp = jnp.pad(x, ((0, N_PAD - N_NODES), (0, 0)))

    p1, q1 = _dense1(xp, W1_l, W1_r, b1.reshape(1, HID_DIM))
    s0, s1, dg0, dg1 = _make_sc_scatter(HID_DIM, True)(p1, src, dst)
    p2, q2, invd = _mid(s0, s1, dg0.reshape(N_PAD, 1), dg1.reshape(N_PAD, 1),
                        q1, W2_l, W2_r, b2.reshape(1, OUT_DIM))
    t0, t1 = _make_sc_scatter(OUT_DIM, False)(p2, src, dst)
    (out,) = _final(t0, t1, invd, q2)
    return out[:N_NODES]


# speedup vs baseline: 2.5770x; 2.5770x over previous
import functools

import jax
import jax.numpy as jnp
from jax import lax
from jax.experimental import pallas as pl
from jax.experimental.pallas import tpu as pltpu
from jax.experimental.pallas import tpu_sc as plsc

N_NODES = 10000
N_EDGES = 160000
IN_DIM = 256
HID_DIM = 128
OUT_DIM = 64

NC, NS = 2, 16
NW = NC * NS
N_PAD = 10240
ROWS_PER_TILE = N_PAD // NS
EB = 128
BLK_PER_W = -(-N_EDGES // (EB * NW))
E_PAD = BLK_PER_W * EB * NW

_TC_ROWS = 1024
_GRID = N_PAD // _TC_ROWS


def _dense1_body(x_ref, wl_ref, wr_ref, b_ref, p_ref, q_ref):
    x = x_ref[...]
    dn = (((1,), (1,)), ((), ()))
    p_ref[...] = lax.dot_general(x, wl_ref[...], dn,
                                 preferred_element_type=jnp.float32)
    q_ref[...] = lax.dot_general(x, wr_ref[...], dn,
                                 preferred_element_type=jnp.float32) + b_ref[...]


@jax.jit
def _dense1(xp, w1l, w1r, b1):
    return pl.pallas_call(
        _dense1_body,
        grid=(_GRID,),
        in_specs=[
            pl.BlockSpec((_TC_ROWS, IN_DIM), lambda i: (i, 0)),
            pl.BlockSpec((HID_DIM, IN_DIM), lambda i: (0, 0)),
            pl.BlockSpec((HID_DIM, IN_DIM), lambda i: (0, 0)),
            pl.BlockSpec((1, HID_DIM), lambda i: (0, 0)),
        ],
        out_specs=[
            pl.BlockSpec((_TC_ROWS, HID_DIM), lambda i: (i, 0)),
            pl.BlockSpec((_TC_ROWS, HID_DIM), lambda i: (i, 0)),
        ],
        out_shape=[
            jax.ShapeDtypeStruct((N_PAD, HID_DIM), jnp.float32),
            jax.ShapeDtypeStruct((N_PAD, HID_DIM), jnp.float32),
        ],
    )(xp, w1l, w1r, b1)


def _mid_body(s0_ref, s1_ref, d0_ref, d1_ref, q1_ref, wl_ref, wr_ref, b_ref,
              p_ref, q_ref, invd_ref):
    deg = jnp.maximum(d0_ref[...] + d1_ref[...], 1.0)
    h = jnp.maximum((s0_ref[...] + s1_ref[...]) / deg + q1_ref[...], 0.0)
    dn = (((1,), (1,)), ((), ()))
    p_ref[...] = lax.dot_general(h, wl_ref[...], dn,
                                 preferred_element_type=jnp.float32)
    q_ref[...] = lax.dot_general(h, wr_ref[...], dn,
                                 preferred_element_type=jnp.float32) + b_ref[...]
    invd_ref[...] = 1.0 / deg


@jax.jit
def _mid(s0, s1, d0, d1, q1, w2l, w2r, b2):
    return pl.pallas_call(
        _mid_body,
        grid=(_GRID,),
        in_specs=[
            pl.BlockSpec((_TC_ROWS, HID_DIM), lambda i: (i, 0)),
            pl.BlockSpec((_TC_ROWS, HID_DIM), lambda i: (i, 0)),
            pl.BlockSpec((_TC_ROWS, 1), lambda i: (i, 0)),
            pl.BlockSpec((_TC_ROWS, 1), lambda i: (i, 0)),
            pl.BlockSpec((_TC_ROWS, HID_DIM), lambda i: (i, 0)),
            pl.BlockSpec((OUT_DIM, HID_DIM), lambda i: (0, 0)),
            pl.BlockSpec((OUT_DIM, HID_DIM), lambda i: (0, 0)),
            pl.BlockSpec((1, OUT_DIM), lambda i: (0, 0)),
        ],
        out_specs=[
            pl.BlockSpec((_TC_ROWS, OUT_DIM), lambda i: (i, 0)),
            pl.BlockSpec((_TC_ROWS, OUT_DIM), lambda i: (i, 0)),
            pl.BlockSpec((_TC_ROWS, 1), lambda i: (i, 0)),
        ],
        out_shape=[
            jax.ShapeDtypeStruct((N_PAD, OUT_DIM), jnp.float32),
            jax.ShapeDtypeStruct((N_PAD, OUT_DIM), jnp.float32),
            jax.ShapeDtypeStruct((N_PAD, 1), jnp.float32),
        ],
    )(s0, s1, d0, d1, q1, w2l, w2r, b2)


def _final_body(t0_ref, t1_ref, invd_ref, q2_ref, o_ref):
    o_ref[...] = jnp.maximum(
        (t0_ref[...] + t1_ref[...]) * invd_ref[...] + q2_ref[...], 0.0)


@jax.jit
def _final(t0, t1, invd, q2):
    return pl.pallas_call(
        _final_body,
        grid=(_GRID,),
        in_specs=[
            pl.BlockSpec((_TC_ROWS, OUT_DIM), lambda i: (i, 0)),
            pl.BlockSpec((_TC_ROWS, OUT_DIM), lambda i: (i, 0)),
            pl.BlockSpec((_TC_ROWS, 1), lambda i: (i, 0)),
            pl.BlockSpec((_TC_ROWS, OUT_DIM), lambda i: (i, 0)),
        ],
        out_specs=[pl.BlockSpec((_TC_ROWS, OUT_DIM), lambda i: (i, 0))],
        out_shape=[jax.ShapeDtypeStruct((N_PAD, OUT_DIM), jnp.float32)],
    )(t0, t1, invd, q2)


@functools.cache
def _make_sc_scatter(d, with_deg):
    mesh = plsc.VectorSubcoreMesh(core_axis_name="c", subcore_axis_name="s",
                                  num_cores=NC, num_subcores=NS)
    out_type = [jax.ShapeDtypeStruct((N_PAD, d), jnp.float32)] * 2
    if with_deg:
        out_type += [jax.ShapeDtypeStruct((N_PAD,), jnp.float32)] * 2
    scratch = [
        pltpu.VMEM((BLK_PER_W, EB), jnp.int32),
        pltpu.VMEM((BLK_PER_W, EB), jnp.int32),
        pltpu.VMEM((EB, d), jnp.float32),
        pltpu.VMEM((EB, d), jnp.float32),
        pltpu.VMEM_SHARED((N_PAD, d), jnp.float32),
        pltpu.SemaphoreType.DMA,
        pltpu.SemaphoreType.DMA,
        pltpu.SemaphoreType.DMA,
        pltpu.SemaphoreType.DMA,
    ]
    if with_deg:
        scratch += [
            pltpu.VMEM((EB,), jnp.float32),
            pltpu.VMEM((128,), jnp.float32),
            pltpu.VMEM_SHARED((N_PAD,), jnp.float32),
            pltpu.SemaphoreType.DMA,
        ]

    def body(p_hbm, src_hbm, dst_hbm, *rest):
        if with_deg:
            (out0, out1, dg0, dg1, srcv, dstv, rows0, rows1, acc,
             gs0, gs1, ss0, ss1, onesv, zv, dacc, dsem) = rest
        else:
            (out0, out1, srcv, dstv, rows0, rows1, acc,
             gs0, gs1, ss0, ss1) = rest
            dg0 = dg1 = onesv = zv = dacc = dsem = None
        cid = lax.axis_index("c")
        sid = lax.axis_index("s")
        wid = sid * NC + cid
        base = sid * ROWS_PER_TILE
        rows = (rows0, rows1)
        gsem = (gs0, gs1)
        ssem = (ss0, ss1)

        pltpu.sync_copy(src_hbm.at[pl.ds(wid * BLK_PER_W, BLK_PER_W)], srcv)
        pltpu.sync_copy(dst_hbm.at[pl.ds(wid * BLK_PER_W, BLK_PER_W)], dstv)

        nvec = d // 16

        def zrow(i, _):
            r = i // nvec
            c = (i % nvec) * 16
            rows0[r, pl.ds(c, 16)] = jnp.zeros((16,), jnp.float32)
            return 0

        lax.fori_loop(0, EB * nvec, zrow, 0)
        for k in range(ROWS_PER_TILE // EB):
            pltpu.sync_copy(rows0, acc.at[pl.ds(base + k * EB, EB)])
        if with_deg:
            def fill(i, _):
                onesv[pl.ds(i * 16, 16)] = jnp.ones((16,), jnp.float32)
                zv[pl.ds(i * 16, 16)] = jnp.zeros((16,), jnp.float32)
                return 0

            lax.fori_loop(0, EB // 16, fill, 0)
            for k in range(ROWS_PER_TILE // 128):
                pltpu.sync_copy(zv, dacc.at[pl.ds(base + k * 128, 128)])
        plsc.subcore_barrier()

        gd = [None, None]
        sd = [None, None]
        dd = None
        gd[0] = pltpu.async_copy(p_hbm.at[srcv.at[0]], rows[0], gsem[0])
        for j in range(BLK_PER_W):
            b = j % 2
            nb = (j + 1) % 2
            if j + 1 < BLK_PER_W:
                if sd[nb] is not None:
                    sd[nb].wait()
                gd[nb] = pltpu.async_copy(p_hbm.at[srcv.at[j + 1]],
                                          rows[nb], gsem[nb])
            gd[b].wait()
            sd[b] = pltpu.async_copy(rows[b], acc.at[dstv.at[j]],
                                     ssem[b], add=True)
            if with_deg:
                if dd is not None:
                    dd.wait()
                dd = pltpu.async_copy(onesv, dacc.at[dstv.at[j]],
                                      dsem, add=True)
        sd[0].wait()
        sd[1].wait()
        if with_deg:
            dd.wait()
        plsc.subcore_barrier()

        @pl.when(cid == 0)
        def _():
            pltpu.sync_copy(acc.at[pl.ds(base, ROWS_PER_TILE)],
                            out0.at[pl.ds(base, ROWS_PER_TILE)])
            if with_deg:
                pltpu.sync_copy(dacc.at[pl.ds(base, ROWS_PER_TILE)],
                                dg0.at[pl.ds(base, ROWS_PER_TILE)])

        @pl.when(cid == 1)
        def _():
            pltpu.sync_copy(acc.at[pl.ds(base, ROWS_PER_TILE)],
                            out1.at[pl.ds(base, ROWS_PER_TILE)])
            if with_deg:
                pltpu.sync_copy(dacc.at[pl.ds(base, ROWS_PER_TILE)],
                                dg1.at[pl.ds(base, ROWS_PER_TILE)])

    return pl.kernel(body, out_type=out_type, mesh=mesh,
                     scratch_types=scratch,
                     compiler_params=pltpu.CompilerParams(
                         use_tc_tiling_on_sc=False))


def kernel(x, edge_index, W1_l, b1, W1_r, W2_l, b2, W2_r):
    npad_e = E_PAD - N_EDGES
    src = jnp.concatenate([edge_index[0].astype(jnp.int32),
                           jnp.arange(npad_e, dtype=jnp.int32) % N_NODES])
    dst = jnp.concatenate([edge_index[1].astype(jnp.int32),
                           N_NODES + (jnp.arange(npad_e, dtype=jnp.int32)
                                      % (N_PAD - N_NODES))])
    src = src.reshape(E_PAD // EB, EB)
    dst = dst.reshape(E_PAD // EB, EB)
    xp = jnp.pad(x, ((0, N_PAD - N_NODES), (0, 0)))

    p1, q1 = _dense1(xp, W1_l, W1_r, b1.reshape(1, HID_DIM))
    s0, s1, dg0, dg1 = _make_sc_scatter(HID_DIM, True)(p1, src, dst)
    p2, q2, invd = _mid(s0, s1, dg0.reshape(N_PAD, 1), dg1.reshape(N_PAD, 1),
                        q1, W2_l, W2_r, b2.reshape(1, OUT_DIM))
    t0, t1 = _make_sc_scatter(OUT_DIM, False)(p2, src, dst)
    (out,) = _final(t0, t1, invd, q2)
    return out[:N_NODES]


# speedup vs baseline: 2.7301x; 1.0594x over previous
import functools

import jax
import jax.numpy as jnp
from jax import lax
from jax.experimental import pallas as pl
from jax.experimental.pallas import tpu as pltpu
from jax.experimental.pallas import tpu_sc as plsc

N_NODES = 10000
N_EDGES = 160000
IN_DIM = 256
HID_DIM = 128
OUT_DIM = 64

NC, NS = 2, 16
NW = NC * NS
N_PAD = 10240
ROWS_PER_TILE = N_PAD // NS
EB = 128
NBLK = N_EDGES // EB
NB_MAIN = NBLK // NW
NW_EXTRA = NBLK - NB_MAIN * NW

_TC_ROWS = 1000
_GRID = N_NODES // _TC_ROWS


def _dense1_body(x_ref, wl_ref, wr_ref, b_ref, p_ref, q_ref):
    x = x_ref[...]
    dn = (((1,), (1,)), ((), ()))
    p_ref[...] = lax.dot_general(x, wl_ref[...], dn,
                                 preferred_element_type=jnp.float32)
    q_ref[...] = lax.dot_general(x, wr_ref[...], dn,
                                 preferred_element_type=jnp.float32) + b_ref[...]


@jax.jit
def _dense1(xp, w1l, w1r, b1):
    return pl.pallas_call(
        _dense1_body,
        grid=(_GRID,),
        in_specs=[
            pl.BlockSpec((_TC_ROWS, IN_DIM), lambda i: (i, 0)),
            pl.BlockSpec((HID_DIM, IN_DIM), lambda i: (0, 0)),
            pl.BlockSpec((HID_DIM, IN_DIM), lambda i: (0, 0)),
            pl.BlockSpec((1, HID_DIM), lambda i: (0, 0)),
        ],
        out_specs=[
            pl.BlockSpec((_TC_ROWS, HID_DIM), lambda i: (i, 0)),
            pl.BlockSpec((_TC_ROWS, HID_DIM), lambda i: (i, 0)),
        ],
        out_shape=[
            jax.ShapeDtypeStruct((N_NODES, HID_DIM), jnp.float32),
            jax.ShapeDtypeStruct((N_NODES, HID_DIM), jnp.float32),
        ],
    )(xp, w1l, w1r, b1)


def _mid_body(s0_ref, s1_ref, d0_ref, d1_ref, q1_ref, wl_ref, wr_ref, b_ref,
              p_ref, q_ref, invd_ref):
    deg = jnp.maximum(d0_ref[...] + d1_ref[...], 1.0)
    h = jnp.maximum((s0_ref[...] + s1_ref[...]) / deg + q1_ref[...], 0.0)
    dn = (((1,), (1,)), ((), ()))
    p_ref[...] = lax.dot_general(h, wl_ref[...], dn,
                                 preferred_element_type=jnp.float32)
    q_ref[...] = lax.dot_general(h, wr_ref[...], dn,
                                 preferred_element_type=jnp.float32) + b_ref[...]
    invd_ref[...] = 1.0 / deg


@jax.jit
def _mid(s0, s1, d0, d1, q1, w2l, w2r, b2):
    return pl.pallas_call(
        _mid_body,
        grid=(_GRID,),
        in_specs=[
            pl.BlockSpec((_TC_ROWS, HID_DIM), lambda i: (i, 0)),
            pl.BlockSpec((_TC_ROWS, HID_DIM), lambda i: (i, 0)),
            pl.BlockSpec((_TC_ROWS, 1), lambda i: (i, 0)),
            pl.BlockSpec((_TC_ROWS, 1), lambda i: (i, 0)),
            pl.BlockSpec((_TC_ROWS, HID_DIM), lambda i: (i, 0)),
            pl.BlockSpec((OUT_DIM, HID_DIM), lambda i: (0, 0)),
            pl.BlockSpec((OUT_DIM, HID_DIM), lambda i: (0, 0)),
            pl.BlockSpec((1, OUT_DIM), lambda i: (0, 0)),
        ],
        out_specs=[
            pl.BlockSpec((_TC_ROWS, OUT_DIM), lambda i: (i, 0)),
            pl.BlockSpec((_TC_ROWS, OUT_DIM), lambda i: (i, 0)),
            pl.BlockSpec((_TC_ROWS, 1), lambda i: (i, 0)),
        ],
        out_shape=[
            jax.ShapeDtypeStruct((N_NODES, OUT_DIM), jnp.float32),
            jax.ShapeDtypeStruct((N_NODES, OUT_DIM), jnp.float32),
            jax.ShapeDtypeStruct((N_NODES, 1), jnp.float32),
        ],
    )(s0, s1, d0, d1, q1, w2l, w2r, b2)


def _final_body(t0_ref, t1_ref, invd_ref, q2_ref, o_ref):
    o_ref[...] = jnp.maximum(
        (t0_ref[...] + t1_ref[...]) * invd_ref[...] + q2_ref[...], 0.0)


@jax.jit
def _final(t0, t1, invd, q2):
    return pl.pallas_call(
        _final_body,
        grid=(_GRID,),
        in_specs=[
            pl.BlockSpec((_TC_ROWS, OUT_DIM), lambda i: (i, 0)),
            pl.BlockSpec((_TC_ROWS, OUT_DIM), lambda i: (i, 0)),
            pl.BlockSpec((_TC_ROWS, 1), lambda i: (i, 0)),
            pl.BlockSpec((_TC_ROWS, OUT_DIM), lambda i: (i, 0)),
        ],
        out_specs=[pl.BlockSpec((_TC_ROWS, OUT_DIM), lambda i: (i, 0))],
        out_shape=[jax.ShapeDtypeStruct((N_NODES, OUT_DIM), jnp.float32)],
    )(t0, t1, invd, q2)


@functools.cache
def _make_sc_scatter(d, with_deg):
    mesh = plsc.VectorSubcoreMesh(core_axis_name="c", subcore_axis_name="s",
                                  num_cores=NC, num_subcores=NS)
    out_type = [jax.ShapeDtypeStruct((N_PAD, d), jnp.float32)] * 2
    if with_deg:
        out_type += [jax.ShapeDtypeStruct((N_PAD,), jnp.float32)] * 2
    scratch = [
        pltpu.VMEM((NB_MAIN + 1, EB), jnp.int32),
        pltpu.VMEM((NB_MAIN + 1, EB), jnp.int32),
        pltpu.VMEM((EB, d), jnp.float32),
        pltpu.VMEM((EB, d), jnp.float32),
        pltpu.VMEM_SHARED((N_PAD, d), jnp.float32),
        pltpu.SemaphoreType.DMA,
        pltpu.SemaphoreType.DMA,
        pltpu.SemaphoreType.DMA,
        pltpu.SemaphoreType.DMA,
    ]
    if with_deg:
        scratch += [
            pltpu.VMEM((EB,), jnp.float32),
            pltpu.VMEM((128,), jnp.float32),
            pltpu.VMEM_SHARED((N_PAD,), jnp.float32),
            pltpu.SemaphoreType.DMA,
        ]

    def body(p_hbm, eidx_hbm, *rest):
        if with_deg:
            (out0, out1, dg0, dg1, srcv, dstv, rows0, rows1, acc,
             gs0, gs1, ss0, ss1, onesv, zv, dacc, dsem) = rest
        else:
            (out0, out1, srcv, dstv, rows0, rows1, acc,
             gs0, gs1, ss0, ss1) = rest
            dg0 = dg1 = onesv = zv = dacc = dsem = None
        cid = lax.axis_index("c")
        sid = lax.axis_index("s")
        wid = sid * NC + cid
        base = sid * ROWS_PER_TILE
        rows = (rows0, rows1)
        gsem = (gs0, gs1)
        ssem = (ss0, ss1)

        pltpu.sync_copy(eidx_hbm.at[0, pl.ds(wid * NB_MAIN, NB_MAIN)],
                        srcv.at[pl.ds(0, NB_MAIN)])
        pltpu.sync_copy(eidx_hbm.at[1, pl.ds(wid * NB_MAIN, NB_MAIN)],
                        dstv.at[pl.ds(0, NB_MAIN)])

        @pl.when(wid < NW_EXTRA)
        def _():
            tail = NB_MAIN * NW + wid
            pltpu.sync_copy(eidx_hbm.at[0, pl.ds(tail, 1)],
                            srcv.at[pl.ds(NB_MAIN, 1)])
            pltpu.sync_copy(eidx_hbm.at[1, pl.ds(tail, 1)],
                            dstv.at[pl.ds(NB_MAIN, 1)])

        nvec = d // 16

        def zrow(i, _):
            r = i // nvec
            c = (i % nvec) * 16
            rows0[r, pl.ds(c, 16)] = jnp.zeros((16,), jnp.float32)
            return 0

        lax.fori_loop(0, EB * nvec, zrow, 0)
        for k in range(ROWS_PER_TILE // EB):
            pltpu.sync_copy(rows0, acc.at[pl.ds(base + k * EB, EB)])
        if with_deg:
            def fill(i, _):
                onesv[pl.ds(i * 16, 16)] = jnp.ones((16,), jnp.float32)
                zv[pl.ds(i * 16, 16)] = jnp.zeros((16,), jnp.float32)
                return 0

            lax.fori_loop(0, EB // 16, fill, 0)
            for k in range(ROWS_PER_TILE // 128):
                pltpu.sync_copy(zv, dacc.at[pl.ds(base + k * 128, 128)])
        plsc.subcore_barrier()

        gd = [None, None]
        sd = [None, None]
        dd = None
        gd[0] = pltpu.async_copy(p_hbm.at[srcv.at[0]], rows[0], gsem[0])
        for j in range(NB_MAIN):
            b = j % 2
            nb = (j + 1) % 2
            if j + 1 < NB_MAIN:
                if sd[nb] is not None:
                    sd[nb].wait()
                gd[nb] = pltpu.async_copy(p_hbm.at[srcv.at[j + 1]],
                                          rows[nb], gsem[nb])
            gd[b].wait()
            sd[b] = pltpu.async_copy(rows[b], acc.at[dstv.at[j]],
                                     ssem[b], add=True)
            if with_deg:
                if dd is not None:
                    dd.wait()
                dd = pltpu.async_copy(onesv, dacc.at[dstv.at[j]],
                                      dsem, add=True)
        sd[0].wait()
        sd[1].wait()
        if with_deg:
            dd.wait()

        @pl.when(wid < NW_EXTRA)
        def _():
            pltpu.async_copy(p_hbm.at[srcv.at[NB_MAIN]], rows0, gs0).wait()
            pltpu.sync_copy(rows0, acc.at[dstv.at[NB_MAIN]], add=True)
            if with_deg:
                pltpu.sync_copy(onesv, dacc.at[dstv.at[NB_MAIN]], add=True)

        plsc.subcore_barrier()

        @pl.when(cid == 0)
        def _():
            pltpu.sync_copy(acc.at[pl.ds(base, ROWS_PER_TILE)],
                            out0.at[pl.ds(base, ROWS_PER_TILE)])
            if with_deg:
                pltpu.sync_copy(dacc.at[pl.ds(base, ROWS_PER_TILE)],
                                dg0.at[pl.ds(base, ROWS_PER_TILE)])

        @pl.when(cid == 1)
        def _():
            pltpu.sync_copy(acc.at[pl.ds(base, ROWS_PER_TILE)],
                            out1.at[pl.ds(base, ROWS_PER_TILE)])
            if with_deg:
                pltpu.sync_copy(dacc.at[pl.ds(base, ROWS_PER_TILE)],
                                dg1.at[pl.ds(base, ROWS_PER_TILE)])

    return pl.kernel(body, out_type=out_type, mesh=mesh,
                     scratch_types=scratch,
                     compiler_params=pltpu.CompilerParams(
                         use_tc_tiling_on_sc=False))


def kernel(x, edge_index, W1_l, b1, W1_r, W2_l, b2, W2_r):
    eidx = edge_index.astype(jnp.int32).reshape(2, NBLK, EB)

    p1, q1 = _dense1(x, W1_l, W1_r, b1.reshape(1, HID_DIM))
    s0, s1, dg0, dg1 = _make_sc_scatter(HID_DIM, True)(p1, eidx)
    p2, q2, invd = _mid(s0, s1, dg0.reshape(N_PAD, 1), dg1.reshape(N_PAD, 1),
                        q1, W2_l, W2_r, b2.reshape(1, OUT_DIM))
    t0, t1 = _make_sc_scatter(OUT_DIM, False)(p2, eidx)
    (out,) = _final(t0, t1, invd, q2)
    return out


# speedup vs baseline: 2.8642x; 1.0491x over previous
import functools

import jax
import jax.numpy as jnp
from jax import lax
from jax.experimental import pallas as pl
from jax.experimental.pallas import tpu as pltpu
from jax.experimental.pallas import tpu_sc as plsc

N_NODES = 10000
N_EDGES = 160000
IN_DIM = 256
HID_DIM = 128
OUT_DIM = 64

NC, NS = 2, 16
NW = NC * NS
N_PAD = 10240
ROWS_PER_TILE = N_PAD // NS
EB = 128
NBLK = N_EDGES // EB
BLKW = -(-NBLK // NW)
NBLK_PAD = BLKW * NW

_TC_ROWS = 1000
_GRID = N_NODES // _TC_ROWS


def _dense1_body(x_ref, wl_ref, wr_ref, b_ref, p_ref, q_ref):
    x = x_ref[...]
    dn = (((1,), (1,)), ((), ()))
    p_ref[...] = lax.dot_general(x, wl_ref[...], dn,
                                 preferred_element_type=jnp.float32)
    q_ref[...] = lax.dot_general(x, wr_ref[...], dn,
                                 preferred_element_type=jnp.float32) + b_ref[...]


@jax.jit
def _dense1(xp, w1l, w1r, b1):
    return pl.pallas_call(
        _dense1_body,
        grid=(_GRID,),
        in_specs=[
            pl.BlockSpec((_TC_ROWS, IN_DIM), lambda i: (i, 0)),
            pl.BlockSpec((HID_DIM, IN_DIM), lambda i: (0, 0)),
            pl.BlockSpec((HID_DIM, IN_DIM), lambda i: (0, 0)),
            pl.BlockSpec((1, HID_DIM), lambda i: (0, 0)),
        ],
        out_specs=[
            pl.BlockSpec((_TC_ROWS, HID_DIM), lambda i: (i, 0)),
            pl.BlockSpec((_TC_ROWS, HID_DIM), lambda i: (i, 0)),
        ],
        out_shape=[
            jax.ShapeDtypeStruct((N_NODES, HID_DIM), jnp.float32),
            jax.ShapeDtypeStruct((N_NODES, HID_DIM), jnp.float32),
        ],
    )(xp, w1l, w1r, b1)


def _mid_body(s0_ref, s1_ref, d0_ref, d1_ref, q1_ref, wl_ref, wr_ref, b_ref,
              p_ref, q_ref, invd_ref):
    deg = jnp.maximum(d0_ref[...] + d1_ref[...], 1.0)
    h = jnp.maximum((s0_ref[...] + s1_ref[...]) / deg + q1_ref[...], 0.0)
    dn = (((1,), (1,)), ((), ()))
    p_ref[...] = lax.dot_general(h, wl_ref[...], dn,
                                 preferred_element_type=jnp.float32)
    q_ref[...] = lax.dot_general(h, wr_ref[...], dn,
                                 preferred_element_type=jnp.float32) + b_ref[...]
    invd_ref[...] = 1.0 / deg


@jax.jit
def _mid(s0, s1, d0, d1, q1, w2l, w2r, b2):
    return pl.pallas_call(
        _mid_body,
        grid=(_GRID,),
        in_specs=[
            pl.BlockSpec((_TC_ROWS, HID_DIM), lambda i: (i, 0)),
            pl.BlockSpec((_TC_ROWS, HID_DIM), lambda i: (i, 0)),
            pl.BlockSpec((_TC_ROWS, 1), lambda i: (i, 0)),
            pl.BlockSpec((_TC_ROWS, 1), lambda i: (i, 0)),
            pl.BlockSpec((_TC_ROWS, HID_DIM), lambda i: (i, 0)),
            pl.BlockSpec((OUT_DIM, HID_DIM), lambda i: (0, 0)),
            pl.BlockSpec((OUT_DIM, HID_DIM), lambda i: (0, 0)),
            pl.BlockSpec((1, OUT_DIM), lambda i: (0, 0)),
        ],
        out_specs=[
            pl.BlockSpec((_TC_ROWS, OUT_DIM), lambda i: (i, 0)),
            pl.BlockSpec((_TC_ROWS, OUT_DIM), lambda i: (i, 0)),
            pl.BlockSpec((_TC_ROWS, 1), lambda i: (i, 0)),
        ],
        out_shape=[
            jax.ShapeDtypeStruct((N_NODES, OUT_DIM), jnp.float32),
            jax.ShapeDtypeStruct((N_NODES, OUT_DIM), jnp.float32),
            jax.ShapeDtypeStruct((N_NODES, 1), jnp.float32),
        ],
    )(s0, s1, d0, d1, q1, w2l, w2r, b2)


def _final_body(t0_ref, t1_ref, invd_ref, q2_ref, o_ref):
    o_ref[...] = jnp.maximum(
        (t0_ref[...] + t1_ref[...]) * invd_ref[...] + q2_ref[...], 0.0)


@jax.jit
def _final(t0, t1, invd, q2):
    return pl.pallas_call(
        _final_body,
        grid=(_GRID,),
        in_specs=[
            pl.BlockSpec((_TC_ROWS, OUT_DIM), lambda i: (i, 0)),
            pl.BlockSpec((_TC_ROWS, OUT_DIM), lambda i: (i, 0)),
            pl.BlockSpec((_TC_ROWS, 1), lambda i: (i, 0)),
            pl.BlockSpec((_TC_ROWS, OUT_DIM), lambda i: (i, 0)),
        ],
        out_specs=[pl.BlockSpec((_TC_ROWS, OUT_DIM), lambda i: (i, 0))],
        out_shape=[jax.ShapeDtypeStruct((N_NODES, OUT_DIM), jnp.float32)],
    )(t0, t1, invd, q2)


@functools.cache
def _make_sc_scatter(d, with_deg, tc_tiling, nbuf):
    mesh = plsc.VectorSubcoreMesh(core_axis_name="c", subcore_axis_name="s",
                                  num_cores=NC, num_subcores=NS)
    out_type = [jax.ShapeDtypeStruct((N_PAD, d), jnp.float32)] * 2
    if with_deg:
        out_type += [jax.ShapeDtypeStruct((N_PAD,), jnp.float32)] * 2
    scratch = (
        [pltpu.VMEM((BLKW, EB), jnp.int32)] * 2
        + [pltpu.VMEM((EB, d), jnp.float32)] * nbuf
        + [pltpu.VMEM_SHARED((N_PAD, d), jnp.float32)]
        + [pltpu.SemaphoreType.DMA] * (2 * nbuf)
    )
    if with_deg:
        scratch += [
            pltpu.VMEM((EB,), jnp.float32),
            pltpu.VMEM((128,), jnp.float32),
            pltpu.VMEM_SHARED((N_PAD,), jnp.float32),
            pltpu.SemaphoreType.DMA,
            pltpu.SemaphoreType.DMA,
        ]

    def body(p_hbm, eidx_hbm, *rest):
        nout = 4 if with_deg else 2
        outs, rest = rest[:nout], rest[nout:]
        srcv, dstv = rest[0], rest[1]
        rows = rest[2:2 + nbuf]
        acc = rest[2 + nbuf]
        gsem = rest[3 + nbuf:3 + 2 * nbuf]
        ssem = rest[3 + 2 * nbuf:3 + 3 * nbuf]
        if with_deg:
            out0, out1, dg0, dg1 = outs
            onesv, zv, dacc, ds0, ds1 = rest[3 + 3 * nbuf:]
            dsem = (ds0, ds1)
        else:
            out0, out1 = outs
        cid = lax.axis_index("c")
        sid = lax.axis_index("s")
        wid = sid * NC + cid
        base = sid * ROWS_PER_TILE

        pltpu.sync_copy(eidx_hbm.at[0, pl.ds(wid * BLKW, BLKW)], srcv)
        pltpu.sync_copy(eidx_hbm.at[1, pl.ds(wid * BLKW, BLKW)], dstv)

        nvec = d // 16

        def zrow(i, _):
            r = i // nvec
            c = (i % nvec) * 16
            rows[0][r, pl.ds(c, 16)] = jnp.zeros((16,), jnp.float32)
            return 0

        lax.fori_loop(0, EB * nvec, zrow, 0)
        for k in range(ROWS_PER_TILE // EB):
            pltpu.sync_copy(rows[0], acc.at[pl.ds(base + k * EB, EB)])
        if with_deg:
            def fill(i, _):
                onesv[pl.ds(i * 16, 16)] = jnp.ones((16,), jnp.float32)
                zv[pl.ds(i * 16, 16)] = jnp.zeros((16,), jnp.float32)
                return 0

            lax.fori_loop(0, EB // 16, fill, 0)
            for k in range(ROWS_PER_TILE // 128):
                pltpu.sync_copy(zv, dacc.at[pl.ds(base + k * 128, 128)])
        plsc.subcore_barrier()

        gd = [None] * nbuf
        sd = [None] * nbuf
        dd = [None, None]
        for j in range(min(nbuf - 1, BLKW)):
            gd[j] = pltpu.async_copy(p_hbm.at[srcv.at[j]], rows[j], gsem[j])
        for j in range(BLKW):
            b = j % nbuf
            f = j + nbuf - 1
            if f < BLKW:
                fb = f % nbuf
                if sd[fb] is not None:
                    sd[fb].wait()
                gd[fb] = pltpu.async_copy(p_hbm.at[srcv.at[f]],
                                          rows[fb], gsem[fb])
            gd[b].wait()
            sd[b] = pltpu.async_copy(rows[b], acc.at[dstv.at[j]],
                                     ssem[b], add=True)
            if with_deg:
                db = j % 2
                if dd[db] is not None:
                    dd[db].wait()
                dd[db] = pltpu.async_copy(onesv, dacc.at[dstv.at[j]],
                                          dsem[db], add=True)
        for desc in (sd + dd) if with_deg else sd:
            if desc is not None:
                desc.wait()
        plsc.subcore_barrier()

        @pl.when(cid == 0)
        def _():
            pltpu.sync_copy(acc.at[pl.ds(base, ROWS_PER_TILE)],
                            out0.at[pl.ds(base, ROWS_PER_TILE)])
            if with_deg:
                pltpu.sync_copy(dacc.at[pl.ds(base, ROWS_PER_TILE)],
                                dg0.at[pl.ds(base, ROWS_PER_TILE)])

        @pl.when(cid == 1)
        def _():
            pltpu.sync_copy(acc.at[pl.ds(base, ROWS_PER_TILE)],
                            out1.at[pl.ds(base, ROWS_PER_TILE)])
            if with_deg:
                pltpu.sync_copy(dacc.at[pl.ds(base, ROWS_PER_TILE)],
                                dg1.at[pl.ds(base, ROWS_PER_TILE)])

    return pl.kernel(body, out_type=out_type, mesh=mesh,
                     scratch_types=scratch,
                     compiler_params=pltpu.CompilerParams(
                         use_tc_tiling_on_sc=tc_tiling))


_PAD_BLOCKS = None


def _pad_blocks():
    global _PAD_BLOCKS
    if _PAD_BLOCKS is None:
        import numpy as np
        n = (NBLK_PAD - NBLK) * EB
        ar = np.arange(n, dtype=np.int32)
        _PAD_BLOCKS = jnp.asarray(np.stack([
            (ar * 7919) % N_NODES,
            N_NODES + ar % (N_PAD - N_NODES),
        ]).reshape(2, NBLK_PAD - NBLK, EB))
    return _PAD_BLOCKS


def kernel(x, edge_index, W1_l, b1, W1_r, W2_l, b2, W2_r):
    eidx = jnp.concatenate(
        [edge_index.astype(jnp.int32).reshape(2, NBLK, EB), _pad_blocks()],
        axis=1)

    p1, q1 = _dense1(x, W1_l, W1_r, b1.reshape(1, HID_DIM))
    s0, s1, dg0, dg1 = _make_sc_scatter(HID_DIM, True, False, 2)(p1, eidx)
    p2, q2, invd = _mid(s0, s1, dg0.reshape(N_PAD, 1), dg1.reshape(N_PAD, 1),
                        q1, W2_l, W2_r, b2.reshape(1, OUT_DIM))
    t0, t1 = _make_sc_scatter(OUT_DIM, False, False, 4)(p2, eidx)
    (out,) = _final(t0, t1, invd, q2)
    return out


# speedup vs baseline: 3.1399x; 1.0963x over previous
import functools

import jax
import jax.numpy as jnp
from jax import lax
from jax.experimental import pallas as pl
from jax.experimental.pallas import tpu as pltpu
from jax.experimental.pallas import tpu_sc as plsc

N_NODES = 10000
N_EDGES = 160000
IN_DIM = 256
HID_DIM = 128
OUT_DIM = 64

NC, NS = 2, 16
NW = NC * NS
N_PAD = 10240
ROWS_PER_TILE = N_PAD // NS
EB = 128
NBLK = N_EDGES // EB
BLKW = -(-NBLK // NW)
NBLK_PAD = BLKW * NW

_TC_ROWS = 1000
_GRID = N_NODES // _TC_ROWS


def _dense1_body(x_ref, wl_ref, wr_ref, b_ref, p_ref, q_ref):
    x = x_ref[...]
    dn = (((1,), (1,)), ((), ()))
    p_ref[...] = lax.dot_general(x, wl_ref[...], dn,
                                 preferred_element_type=jnp.float32
                                 ).astype(jnp.bfloat16)
    q_ref[...] = lax.dot_general(x, wr_ref[...], dn,
                                 preferred_element_type=jnp.float32) + b_ref[...]


@jax.jit
def _dense1(xp, w1l, w1r, b1):
    return pl.pallas_call(
        _dense1_body,
        grid=(_GRID,),
        in_specs=[
            pl.BlockSpec((_TC_ROWS, IN_DIM), lambda i: (i, 0)),
            pl.BlockSpec((HID_DIM, IN_DIM), lambda i: (0, 0)),
            pl.BlockSpec((HID_DIM, IN_DIM), lambda i: (0, 0)),
            pl.BlockSpec((1, HID_DIM), lambda i: (0, 0)),
        ],
        out_specs=[
            pl.BlockSpec((_TC_ROWS, HID_DIM), lambda i: (i, 0)),
            pl.BlockSpec((_TC_ROWS, HID_DIM), lambda i: (i, 0)),
        ],
        out_shape=[
            jax.ShapeDtypeStruct((N_NODES, HID_DIM), jnp.bfloat16),
            jax.ShapeDtypeStruct((N_NODES, HID_DIM), jnp.float32),
        ],
    )(xp, w1l, w1r, b1)


def _mid_body(s0_ref, s1_ref, d0_ref, d1_ref, q1_ref, wl_ref, wr_ref, b_ref,
              p_ref, q_ref, invd_ref):
    deg = jnp.maximum(d0_ref[...] + d1_ref[...], 1.0)
    s = s0_ref[...].astype(jnp.float32) + s1_ref[...].astype(jnp.float32)
    h = jnp.maximum(s / deg + q1_ref[...], 0.0)
    dn = (((1,), (1,)), ((), ()))
    p_ref[...] = lax.dot_general(h, wl_ref[...], dn,
                                 preferred_element_type=jnp.float32
                                 ).astype(jnp.bfloat16)
    q_ref[...] = lax.dot_general(h, wr_ref[...], dn,
                                 preferred_element_type=jnp.float32) + b_ref[...]
    invd_ref[...] = 1.0 / deg


@jax.jit
def _mid(s0, s1, d0, d1, q1, w2l, w2r, b2):
    return pl.pallas_call(
        _mid_body,
        grid=(_GRID,),
        in_specs=[
            pl.BlockSpec((_TC_ROWS, HID_DIM), lambda i: (i, 0)),
            pl.BlockSpec((_TC_ROWS, HID_DIM), lambda i: (i, 0)),
            pl.BlockSpec((_TC_ROWS, 1), lambda i: (i, 0)),
            pl.BlockSpec((_TC_ROWS, 1), lambda i: (i, 0)),
            pl.BlockSpec((_TC_ROWS, HID_DIM), lambda i: (i, 0)),
            pl.BlockSpec((OUT_DIM, HID_DIM), lambda i: (0, 0)),
            pl.BlockSpec((OUT_DIM, HID_DIM), lambda i: (0, 0)),
            pl.BlockSpec((1, OUT_DIM), lambda i: (0, 0)),
        ],
        out_specs=[
            pl.BlockSpec((_TC_ROWS, OUT_DIM), lambda i: (i, 0)),
            pl.BlockSpec((_TC_ROWS, OUT_DIM), lambda i: (i, 0)),
            pl.BlockSpec((_TC_ROWS, 1), lambda i: (i, 0)),
        ],
        out_shape=[
            jax.ShapeDtypeStruct((N_NODES, OUT_DIM), jnp.bfloat16),
            jax.ShapeDtypeStruct((N_NODES, OUT_DIM), jnp.float32),
            jax.ShapeDtypeStruct((N_NODES, 1), jnp.float32),
        ],
    )(s0, s1, d0, d1, q1, w2l, w2r, b2)


def _final_body(t0_ref, t1_ref, invd_ref, q2_ref, o_ref):
    t = t0_ref[...].astype(jnp.float32) + t1_ref[...].astype(jnp.float32)
    o_ref[...] = jnp.maximum(t * invd_ref[...] + q2_ref[...], 0.0)


@jax.jit
def _final(t0, t1, invd, q2):
    return pl.pallas_call(
        _final_body,
        grid=(_GRID,),
        in_specs=[
            pl.BlockSpec((_TC_ROWS, OUT_DIM), lambda i: (i, 0)),
            pl.BlockSpec((_TC_ROWS, OUT_DIM), lambda i: (i, 0)),
            pl.BlockSpec((_TC_ROWS, 1), lambda i: (i, 0)),
            pl.BlockSpec((_TC_ROWS, OUT_DIM), lambda i: (i, 0)),
        ],
        out_specs=[pl.BlockSpec((_TC_ROWS, OUT_DIM), lambda i: (i, 0))],
        out_shape=[jax.ShapeDtypeStruct((N_NODES, OUT_DIM), jnp.float32)],
    )(t0, t1, invd, q2)


@functools.cache
def _make_sc_scatter(d, with_deg, tc_tiling, nbuf):
    mesh = plsc.VectorSubcoreMesh(core_axis_name="c", subcore_axis_name="s",
                                  num_cores=NC, num_subcores=NS)
    out_type = [jax.ShapeDtypeStruct((N_PAD, d), jnp.bfloat16)] * 2
    if with_deg:
        out_type += [jax.ShapeDtypeStruct((N_PAD,), jnp.float32)] * 2
    scratch = (
        [pltpu.VMEM((BLKW, EB), jnp.int32)] * 2
        + [pltpu.VMEM((EB, d), jnp.bfloat16)] * nbuf
        + [pltpu.VMEM_SHARED((N_PAD, d), jnp.bfloat16)]
        + [pltpu.SemaphoreType.DMA] * (2 * nbuf)
    )
    if with_deg:
        scratch += [
            pltpu.VMEM((EB,), jnp.float32),
            pltpu.VMEM((128,), jnp.float32),
            pltpu.VMEM_SHARED((N_PAD,), jnp.float32),
            pltpu.SemaphoreType.DMA,
            pltpu.SemaphoreType.DMA,
        ]

    def body(p_hbm, eidx_hbm, *rest):
        nout = 4 if with_deg else 2
        outs, rest = rest[:nout], rest[nout:]
        srcv, dstv = rest[0], rest[1]
        rows = rest[2:2 + nbuf]
        acc = rest[2 + nbuf]
        gsem = rest[3 + nbuf:3 + 2 * nbuf]
        ssem = rest[3 + 2 * nbuf:3 + 3 * nbuf]
        if with_deg:
            out0, out1, dg0, dg1 = outs
            onesv, zv, dacc, ds0, ds1 = rest[3 + 3 * nbuf:]
            dsem = (ds0, ds1)
        else:
            out0, out1 = outs
        cid = lax.axis_index("c")
        sid = lax.axis_index("s")
        wid = sid * NC + cid
        base = sid * ROWS_PER_TILE

        pltpu.sync_copy(eidx_hbm.at[0, pl.ds(wid * BLKW, BLKW)], srcv)
        pltpu.sync_copy(eidx_hbm.at[1, pl.ds(wid * BLKW, BLKW)], dstv)

        nvec = d // 32

        def zrow(i, _):
            r = i // nvec
            c = (i % nvec) * 32
            rows[0][r, pl.ds(c, 32)] = jnp.zeros((32,), jnp.bfloat16)
            return 0

        lax.fori_loop(0, EB * nvec, zrow, 0)
        for k in range(ROWS_PER_TILE // EB):
            pltpu.sync_copy(rows[0], acc.at[pl.ds(base + k * EB, EB)])
        if with_deg:
            def fill(i, _):
                onesv[pl.ds(i * 16, 16)] = jnp.ones((16,), jnp.float32)
                zv[pl.ds(i * 16, 16)] = jnp.zeros((16,), jnp.float32)
                return 0

            lax.fori_loop(0, EB // 16, fill, 0)
            for k in range(ROWS_PER_TILE // 128):
                pltpu.sync_copy(zv, dacc.at[pl.ds(base + k * 128, 128)])
        plsc.subcore_barrier()

        gd = [None] * nbuf
        sd = [None] * nbuf
        dd = [None, None]
        for j in range(min(nbuf - 1, BLKW)):
            gd[j] = pltpu.async_copy(p_hbm.at[srcv.at[j]], rows[j], gsem[j])
        for j in range(BLKW):
            b = j % nbuf
            f = j + nbuf - 1
            if f < BLKW:
                fb = f % nbuf
                if sd[fb] is not None:
                    sd[fb].wait()
                gd[fb] = pltpu.async_copy(p_hbm.at[srcv.at[f]],
                                          rows[fb], gsem[fb])
            gd[b].wait()
            sd[b] = pltpu.async_copy(rows[b], acc.at[dstv.at[j]],
                                     ssem[b], add=True)
            if with_deg:
                db = j % 2
                if dd[db] is not None:
                    dd[db].wait()
                dd[db] = pltpu.async_copy(onesv, dacc.at[dstv.at[j]],
                                          dsem[db], add=True)
        for desc in (sd + dd) if with_deg else sd:
            if desc is not None:
                desc.wait()
        plsc.subcore_barrier()

        @pl.when(cid == 0)
        def _():
            pltpu.sync_copy(acc.at[pl.ds(base, ROWS_PER_TILE)],
                            out0.at[pl.ds(base, ROWS_PER_TILE)])
            if with_deg:
                pltpu.sync_copy(dacc.at[pl.ds(base, ROWS_PER_TILE)],
                                dg0.at[pl.ds(base, ROWS_PER_TILE)])

        @pl.when(cid == 1)
        def _():
            pltpu.sync_copy(acc.at[pl.ds(base, ROWS_PER_TILE)],
                            out1.at[pl.ds(base, ROWS_PER_TILE)])
            if with_deg:
                pltpu.sync_copy(dacc.at[pl.ds(base, ROWS_PER_TILE)],
                                dg1.at[pl.ds(base, ROWS_PER_TILE)])

    return pl.kernel(body, out_type=out_type, mesh=mesh,
                     scratch_types=scratch,
                     compiler_params=pltpu.CompilerParams(
                         use_tc_tiling_on_sc=tc_tiling))


_PAD_BLOCKS = None


def _pad_blocks():
    global _PAD_BLOCKS
    if _PAD_BLOCKS is None:
        import numpy as np
        n = (NBLK_PAD - NBLK) * EB
        ar = np.arange(n, dtype=np.int32)
        _PAD_BLOCKS = jnp.asarray(np.stack([
            (ar * 7919) % N_NODES,
            N_NODES + ar % (N_PAD - N_NODES),
        ]).reshape(2, NBLK_PAD - NBLK, EB))
    return _PAD_BLOCKS


def kernel(x, edge_index, W1_l, b1, W1_r, W2_l, b2, W2_r):
    eidx = jnp.concatenate(
        [edge_index.astype(jnp.int32).reshape(2, NBLK, EB), _pad_blocks()],
        axis=1)

    p1, q1 = _dense1(x, W1_l, W1_r, b1.reshape(1, HID_DIM))
    s0, s1, dg0, dg1 = _make_sc_scatter(HID_DIM, True, False, 4)(p1, eidx)
    p2, q2, invd = _mid(s0, s1, dg0.reshape(N_PAD, 1), dg1.reshape(N_PAD, 1),
                        q1, W2_l, W2_r, b2.reshape(1, OUT_DIM))
    t0, t1 = _make_sc_scatter(OUT_DIM, False, False, 4)(p2, eidx)
    (out,) = _final(t0, t1, invd, q2)
    return out


# speedup vs baseline: 3.2982x; 1.0504x over previous
import functools

import jax
import jax.numpy as jnp
from jax import lax
from jax.experimental import pallas as pl
from jax.experimental.pallas import tpu as pltpu
from jax.experimental.pallas import tpu_sc as plsc

N_NODES = 10000
N_EDGES = 160000
IN_DIM = 256
HID_DIM = 128
OUT_DIM = 64

NC, NS = 2, 16
NW = NC * NS
N_PAD = 10240
ROWS_PER_TILE = N_PAD // NS
EB = 128
NBLK = N_EDGES // EB
BLKW = -(-NBLK // NW)
NBLK_PAD = BLKW * NW

_TC_ROWS = 2000
_GRID = N_NODES // _TC_ROWS


def _dense1_body(x_ref, wl_ref, wr_ref, b_ref, p_ref, q_ref):
    x = x_ref[...]
    dn = (((1,), (1,)), ((), ()))
    p_ref[...] = lax.dot_general(x, wl_ref[...], dn,
                                 preferred_element_type=jnp.float32
                                 ).astype(jnp.bfloat16)
    q_ref[...] = lax.dot_general(x, wr_ref[...], dn,
                                 preferred_element_type=jnp.float32) + b_ref[...]


@jax.jit
def _dense1(xp, w1l, w1r, b1):
    return pl.pallas_call(
        _dense1_body,
        grid=(_GRID,),
        in_specs=[
            pl.BlockSpec((_TC_ROWS, IN_DIM), lambda i: (i, 0)),
            pl.BlockSpec((HID_DIM, IN_DIM), lambda i: (0, 0)),
            pl.BlockSpec((HID_DIM, IN_DIM), lambda i: (0, 0)),
            pl.BlockSpec((1, HID_DIM), lambda i: (0, 0)),
        ],
        out_specs=[
            pl.BlockSpec((_TC_ROWS, HID_DIM), lambda i: (i, 0)),
            pl.BlockSpec((_TC_ROWS, HID_DIM), lambda i: (i, 0)),
        ],
        out_shape=[
            jax.ShapeDtypeStruct((N_NODES, HID_DIM), jnp.bfloat16),
            jax.ShapeDtypeStruct((N_NODES, HID_DIM), jnp.float32),
        ],
    )(xp, w1l, w1r, b1)


def _mid_body(s0_ref, s1_ref, d0_ref, d1_ref, q1_ref, wl_ref, wr_ref, b_ref,
              p_ref, q_ref, invd_ref):
    deg = jnp.maximum(d0_ref[...] + d1_ref[...], 1.0)
    s = s0_ref[...].astype(jnp.float32) + s1_ref[...].astype(jnp.float32)
    h = jnp.maximum(s / deg + q1_ref[...], 0.0)
    dn = (((1,), (1,)), ((), ()))
    p_ref[...] = lax.dot_general(h, wl_ref[...], dn,
                                 preferred_element_type=jnp.float32
                                 ).astype(jnp.bfloat16)
    q_ref[...] = lax.dot_general(h, wr_ref[...], dn,
                                 preferred_element_type=jnp.float32) + b_ref[...]
    invd_ref[...] = 1.0 / deg


@jax.jit
def _mid(s0, s1, d0, d1, q1, w2l, w2r, b2):
    return pl.pallas_call(
        _mid_body,
        grid=(_GRID,),
        in_specs=[
            pl.BlockSpec((_TC_ROWS, HID_DIM), lambda i: (i, 0)),
            pl.BlockSpec((_TC_ROWS, HID_DIM), lambda i: (i, 0)),
            pl.BlockSpec((_TC_ROWS, 1), lambda i: (i, 0)),
            pl.BlockSpec((_TC_ROWS, 1), lambda i: (i, 0)),
            pl.BlockSpec((_TC_ROWS, HID_DIM), lambda i: (i, 0)),
            pl.BlockSpec((OUT_DIM, HID_DIM), lambda i: (0, 0)),
            pl.BlockSpec((OUT_DIM, HID_DIM), lambda i: (0, 0)),
            pl.BlockSpec((1, OUT_DIM), lambda i: (0, 0)),
        ],
        out_specs=[
            pl.BlockSpec((_TC_ROWS, OUT_DIM), lambda i: (i, 0)),
            pl.BlockSpec((_TC_ROWS, OUT_DIM), lambda i: (i, 0)),
            pl.BlockSpec((_TC_ROWS, 1), lambda i: (i, 0)),
        ],
        out_shape=[
            jax.ShapeDtypeStruct((N_NODES, OUT_DIM), jnp.bfloat16),
            jax.ShapeDtypeStruct((N_NODES, OUT_DIM), jnp.float32),
            jax.ShapeDtypeStruct((N_NODES, 1), jnp.float32),
        ],
    )(s0, s1, d0, d1, q1, w2l, w2r, b2)


def _final_body(t0_ref, t1_ref, invd_ref, q2_ref, o_ref):
    t = t0_ref[...].astype(jnp.float32) + t1_ref[...].astype(jnp.float32)
    o_ref[...] = jnp.maximum(t * invd_ref[...] + q2_ref[...], 0.0)


@jax.jit
def _final(t0, t1, invd, q2):
    return pl.pallas_call(
        _final_body,
        grid=(_GRID,),
        in_specs=[
            pl.BlockSpec((_TC_ROWS, OUT_DIM), lambda i: (i, 0)),
            pl.BlockSpec((_TC_ROWS, OUT_DIM), lambda i: (i, 0)),
            pl.BlockSpec((_TC_ROWS, 1), lambda i: (i, 0)),
            pl.BlockSpec((_TC_ROWS, OUT_DIM), lambda i: (i, 0)),
        ],
        out_specs=[pl.BlockSpec((_TC_ROWS, OUT_DIM), lambda i: (i, 0))],
        out_shape=[jax.ShapeDtypeStruct((N_NODES, OUT_DIM), jnp.float32)],
    )(t0, t1, invd, q2)


@functools.cache
def _make_sc_scatter(d, with_deg, tc_tiling, nbuf):
    mesh = plsc.VectorSubcoreMesh(core_axis_name="c", subcore_axis_name="s",
                                  num_cores=NC, num_subcores=NS)
    out_type = [jax.ShapeDtypeStruct((N_PAD, d), jnp.bfloat16)] * 2
    if with_deg:
        out_type += [jax.ShapeDtypeStruct((N_PAD,), jnp.float32)] * 2
    scratch = (
        [pltpu.VMEM((BLKW, EB), jnp.int32)] * 2
        + [pltpu.VMEM((EB, d), jnp.bfloat16)] * nbuf
        + [pltpu.VMEM_SHARED((N_PAD, d), jnp.bfloat16)]
        + [pltpu.SemaphoreType.DMA] * (2 * nbuf)
    )
    if with_deg:
        scratch += [
            pltpu.VMEM((EB,), jnp.float32),
            pltpu.VMEM((128,), jnp.float32),
            pltpu.VMEM_SHARED((N_PAD,), jnp.float32),
            pltpu.SemaphoreType.DMA,
            pltpu.SemaphoreType.DMA,
        ]

    def body(p_hbm, eidx_hbm, *rest):
        nout = 4 if with_deg else 2
        outs, rest = rest[:nout], rest[nout:]
        srcv, dstv = rest[0], rest[1]
        rows = rest[2:2 + nbuf]
        acc = rest[2 + nbuf]
        gsem = rest[3 + nbuf:3 + 2 * nbuf]
        ssem = rest[3 + 2 * nbuf:3 + 3 * nbuf]
        if with_deg:
            out0, out1, dg0, dg1 = outs
            onesv, zv, dacc, ds0, ds1 = rest[3 + 3 * nbuf:]
            dsem = (ds0, ds1)
        else:
            out0, out1 = outs
        cid = lax.axis_index("c")
        sid = lax.axis_index("s")
        wid = sid * NC + cid
        base = sid * ROWS_PER_TILE

        pltpu.sync_copy(eidx_hbm.at[0, pl.ds(wid * BLKW, BLKW)], srcv)
        pltpu.sync_copy(eidx_hbm.at[1, pl.ds(wid * BLKW, BLKW)], dstv)

        nvec = d // 32

        def zrow(i, _):
            r = i // nvec
            c = (i % nvec) * 32
            rows[0][r, pl.ds(c, 32)] = jnp.zeros((32,), jnp.bfloat16)
            return 0

        lax.fori_loop(0, EB * nvec, zrow, 0)
        for k in range(ROWS_PER_TILE // EB):
            pltpu.sync_copy(rows[0], acc.at[pl.ds(base + k * EB, EB)])
        if with_deg:
            def fill(i, _):
                onesv[pl.ds(i * 16, 16)] = jnp.ones((16,), jnp.float32)
                zv[pl.ds(i * 16, 16)] = jnp.zeros((16,), jnp.float32)
                return 0

            lax.fori_loop(0, EB // 16, fill, 0)
            for k in range(ROWS_PER_TILE // 128):
                pltpu.sync_copy(zv, dacc.at[pl.ds(base + k * 128, 128)])
        plsc.subcore_barrier()

        gd = [None] * nbuf
        sd = [None] * nbuf
        dd = [None, None]
        for j in range(min(nbuf - 1, BLKW)):
            gd[j] = pltpu.async_copy(p_hbm.at[srcv.at[j]], rows[j], gsem[j])
        for j in range(BLKW):
            b = j % nbuf
            f = j + nbuf - 1
            if f < BLKW:
                fb = f % nbuf
                if sd[fb] is not None:
                    sd[fb].wait()
                gd[fb] = pltpu.async_copy(p_hbm.at[srcv.at[f]],
                                          rows[fb], gsem[fb])
            gd[b].wait()
            sd[b] = pltpu.async_copy(rows[b], acc.at[dstv.at[j]],
                                     ssem[b], add=True)
            if with_deg:
                db = j % 2
                if dd[db] is not None:
                    dd[db].wait()
                dd[db] = pltpu.async_copy(onesv, dacc.at[dstv.at[j]],
                                          dsem[db], add=True)
        for desc in (sd + dd) if with_deg else sd:
            if desc is not None:
                desc.wait()
        plsc.subcore_barrier()

        @pl.when(cid == 0)
        def _():
            pltpu.sync_copy(acc.at[pl.ds(base, ROWS_PER_TILE)],
                            out0.at[pl.ds(base, ROWS_PER_TILE)])
            if with_deg:
                pltpu.sync_copy(dacc.at[pl.ds(base, ROWS_PER_TILE)],
                                dg0.at[pl.ds(base, ROWS_PER_TILE)])

        @pl.when(cid == 1)
        def _():
            pltpu.sync_copy(acc.at[pl.ds(base, ROWS_PER_TILE)],
                            out1.at[pl.ds(base, ROWS_PER_TILE)])
            if with_deg:
                pltpu.sync_copy(dacc.at[pl.ds(base, ROWS_PER_TILE)],
                                dg1.at[pl.ds(base, ROWS_PER_TILE)])

    return pl.kernel(body, out_type=out_type, mesh=mesh,
                     scratch_types=scratch,
                     compiler_params=pltpu.CompilerParams(
                         use_tc_tiling_on_sc=tc_tiling))


_PAD_BLOCKS = None


def _pad_blocks():
    global _PAD_BLOCKS
    if _PAD_BLOCKS is None:
        import numpy as np
        n = (NBLK_PAD - NBLK) * EB
        ar = np.arange(n, dtype=np.int32)
        _PAD_BLOCKS = jnp.asarray(np.stack([
            (ar * 7919) % N_NODES,
            N_NODES + ar % (N_PAD - N_NODES),
        ]).reshape(2, NBLK_PAD - NBLK, EB))
    return _PAD_BLOCKS


def kernel(x, edge_index, W1_l, b1, W1_r, W2_l, b2, W2_r):
    eidx = jnp.concatenate(
        [edge_index.astype(jnp.int32).reshape(2, NBLK, EB), _pad_blocks()],
        axis=1)

    p1, q1 = _dense1(x, W1_l, W1_r, b1.reshape(1, HID_DIM))
    s0, s1, dg0, dg1 = _make_sc_scatter(HID_DIM, True, False, 4)(p1, eidx)
    p2, q2, invd = _mid(s0, s1, dg0.reshape(N_PAD, 1), dg1.reshape(N_PAD, 1),
                        q1, W2_l, W2_r, b2.reshape(1, OUT_DIM))
    t0, t1 = _make_sc_scatter(OUT_DIM, False, False, 6)(p2, eidx)
    (out,) = _final(t0, t1, invd, q2)
    return out


# speedup vs baseline: 3.3008x; 1.0008x over previous
import functools

import jax
import jax.numpy as jnp
from jax import lax
from jax.experimental import pallas as pl
from jax.experimental.pallas import tpu as pltpu
from jax.experimental.pallas import tpu_sc as plsc

N_NODES = 10000
N_EDGES = 160000
IN_DIM = 256
HID_DIM = 128
OUT_DIM = 64

NC, NS = 2, 16
NW = NC * NS
N_PAD = 10240
ROWS_PER_TILE = N_PAD // NS
EB = 128
NBLK = N_EDGES // EB
BLKW = -(-NBLK // NW)
NBLK_PAD = BLKW * NW

_TC_ROWS = 2000
_GRID = N_NODES // _TC_ROWS


def _dense1_body(x_ref, wl_ref, wr_ref, b_ref, p_ref, q_ref):
    x = x_ref[...]
    dn = (((1,), (1,)), ((), ()))
    p_ref[...] = lax.dot_general(x, wl_ref[...], dn,
                                 preferred_element_type=jnp.float32
                                 ).astype(jnp.bfloat16)
    q_ref[...] = lax.dot_general(x, wr_ref[...], dn,
                                 preferred_element_type=jnp.float32) + b_ref[...]


@jax.jit
def _dense1(xp, w1l, w1r, b1):
    return pl.pallas_call(
        _dense1_body,
        grid=(_GRID,),
        in_specs=[
            pl.BlockSpec((_TC_ROWS, IN_DIM), lambda i: (i, 0)),
            pl.BlockSpec((HID_DIM, IN_DIM), lambda i: (0, 0)),
            pl.BlockSpec((HID_DIM, IN_DIM), lambda i: (0, 0)),
            pl.BlockSpec((1, HID_DIM), lambda i: (0, 0)),
        ],
        out_specs=[
            pl.BlockSpec((_TC_ROWS, HID_DIM), lambda i: (i, 0)),
            pl.BlockSpec((_TC_ROWS, HID_DIM), lambda i: (i, 0)),
        ],
        out_shape=[
            jax.ShapeDtypeStruct((N_NODES, HID_DIM), jnp.bfloat16),
            jax.ShapeDtypeStruct((N_NODES, HID_DIM), jnp.float32),
        ],
    )(xp, w1l, w1r, b1)


def _mid_body(s0_ref, s1_ref, d0_ref, d1_ref, q1_ref, wl_ref, wr_ref, b_ref,
              p_ref, q_ref, invd_ref):
    deg = jnp.maximum(d0_ref[...] + d1_ref[...], 1.0)
    s = s0_ref[...].astype(jnp.float32) + s1_ref[...].astype(jnp.float32)
    h = jnp.maximum(s / deg + q1_ref[...], 0.0)
    dn = (((1,), (1,)), ((), ()))
    p_ref[...] = lax.dot_general(h, wl_ref[...], dn,
                                 preferred_element_type=jnp.float32
                                 ).astype(jnp.bfloat16)
    q_ref[...] = lax.dot_general(h, wr_ref[...], dn,
                                 preferred_element_type=jnp.float32) + b_ref[...]
    invd_ref[...] = 1.0 / deg


@jax.jit
def _mid(s0, s1, d0, d1, q1, w2l, w2r, b2):
    return pl.pallas_call(
        _mid_body,
        grid=(_GRID,),
        in_specs=[
            pl.BlockSpec((_TC_ROWS, HID_DIM), lambda i: (i, 0)),
            pl.BlockSpec((_TC_ROWS, HID_DIM), lambda i: (i, 0)),
            pl.BlockSpec((_TC_ROWS, 1), lambda i: (i, 0)),
            pl.BlockSpec((_TC_ROWS, 1), lambda i: (i, 0)),
            pl.BlockSpec((_TC_ROWS, HID_DIM), lambda i: (i, 0)),
            pl.BlockSpec((OUT_DIM, HID_DIM), lambda i: (0, 0)),
            pl.BlockSpec((OUT_DIM, HID_DIM), lambda i: (0, 0)),
            pl.BlockSpec((1, OUT_DIM), lambda i: (0, 0)),
        ],
        out_specs=[
            pl.BlockSpec((_TC_ROWS, OUT_DIM), lambda i: (i, 0)),
            pl.BlockSpec((_TC_ROWS, OUT_DIM), lambda i: (i, 0)),
            pl.BlockSpec((_TC_ROWS, 1), lambda i: (i, 0)),
        ],
        out_shape=[
            jax.ShapeDtypeStruct((N_NODES, OUT_DIM), jnp.bfloat16),
            jax.ShapeDtypeStruct((N_NODES, OUT_DIM), jnp.float32),
            jax.ShapeDtypeStruct((N_NODES, 1), jnp.float32),
        ],
    )(s0, s1, d0, d1, q1, w2l, w2r, b2)


def _final_body(t0_ref, t1_ref, invd_ref, q2_ref, o_ref):
    t = t0_ref[...].astype(jnp.float32) + t1_ref[...].astype(jnp.float32)
    o_ref[...] = jnp.maximum(t * invd_ref[...] + q2_ref[...], 0.0)


@jax.jit
def _final(t0, t1, invd, q2):
    return pl.pallas_call(
        _final_body,
        grid=(_GRID,),
        in_specs=[
            pl.BlockSpec((_TC_ROWS, OUT_DIM), lambda i: (i, 0)),
            pl.BlockSpec((_TC_ROWS, OUT_DIM), lambda i: (i, 0)),
            pl.BlockSpec((_TC_ROWS, 1), lambda i: (i, 0)),
            pl.BlockSpec((_TC_ROWS, OUT_DIM), lambda i: (i, 0)),
        ],
        out_specs=[pl.BlockSpec((_TC_ROWS, OUT_DIM), lambda i: (i, 0))],
        out_shape=[jax.ShapeDtypeStruct((N_NODES, OUT_DIM), jnp.float32)],
    )(t0, t1, invd, q2)


@functools.cache
def _make_sc_scatter(d, with_deg, tc_tiling, nbuf):
    mesh = plsc.VectorSubcoreMesh(core_axis_name="c", subcore_axis_name="s",
                                  num_cores=NC, num_subcores=NS)
    out_type = [jax.ShapeDtypeStruct((N_PAD, d), jnp.bfloat16)] * 2
    if with_deg:
        out_type += [jax.ShapeDtypeStruct((N_PAD,), jnp.float32)] * 2
    scratch = (
        [pltpu.VMEM((BLKW, EB), jnp.int32)] * 2
        + [pltpu.VMEM((EB, d), jnp.bfloat16)] * nbuf
        + [pltpu.VMEM_SHARED((N_PAD, d), jnp.bfloat16)]
        + [pltpu.SemaphoreType.DMA] * (2 * nbuf)
    )
    if with_deg:
        scratch += [
            pltpu.VMEM((EB,), jnp.float32),
            pltpu.VMEM((128,), jnp.float32),
            pltpu.VMEM_SHARED((N_PAD,), jnp.float32),
            pltpu.SemaphoreType.DMA,
            pltpu.SemaphoreType.DMA,
        ]

    def body(p_hbm, eidx_hbm, *rest):
        nout = 4 if with_deg else 2
        outs, rest = rest[:nout], rest[nout:]
        srcv, dstv = rest[0], rest[1]
        rows = rest[2:2 + nbuf]
        acc = rest[2 + nbuf]
        gsem = rest[3 + nbuf:3 + 2 * nbuf]
        ssem = rest[3 + 2 * nbuf:3 + 3 * nbuf]
        if with_deg:
            out0, out1, dg0, dg1 = outs
            onesv, zv, dacc, ds0, ds1 = rest[3 + 3 * nbuf:]
            dsem = (ds0, ds1)
        else:
            out0, out1 = outs
        cid = lax.axis_index("c")
        sid = lax.axis_index("s")
        wid = sid * NC + cid
        base = sid * ROWS_PER_TILE

        pltpu.sync_copy(eidx_hbm.at[0, pl.ds(wid * BLKW, BLKW)], srcv)
        pltpu.sync_copy(eidx_hbm.at[1, pl.ds(wid * BLKW, BLKW)], dstv)

        nvec = d // 32

        def zrow(i, _):
            r = i // nvec
            c = (i % nvec) * 32
            rows[0][r, pl.ds(c, 32)] = jnp.zeros((32,), jnp.bfloat16)
            return 0

        lax.fori_loop(0, EB * nvec, zrow, 0)
        for k in range(ROWS_PER_TILE // EB):
            pltpu.sync_copy(rows[0], acc.at[pl.ds(base + k * EB, EB)])
        if with_deg:
            def fill(i, _):
                onesv[pl.ds(i * 16, 16)] = jnp.ones((16,), jnp.float32)
                zv[pl.ds(i * 16, 16)] = jnp.zeros((16,), jnp.float32)
                return 0

            lax.fori_loop(0, EB // 16, fill, 0)
            for k in range(ROWS_PER_TILE // 128):
                pltpu.sync_copy(zv, dacc.at[pl.ds(base + k * 128, 128)])
        plsc.subcore_barrier()

        gd = [None] * nbuf
        sd = [None] * nbuf
        dd = [None, None]
        for j in range(min(nbuf - 1, BLKW)):
            gd[j] = pltpu.async_copy(p_hbm.at[srcv.at[j]], rows[j], gsem[j])
        for j in range(BLKW):
            b = j % nbuf
            f = j + nbuf - 1
            if f < BLKW:
                fb = f % nbuf
                if sd[fb] is not None:
                    sd[fb].wait()
                gd[fb] = pltpu.async_copy(p_hbm.at[srcv.at[f]],
                                          rows[fb], gsem[fb])
            gd[b].wait()
            sd[b] = pltpu.async_copy(rows[b], acc.at[dstv.at[j]],
                                     ssem[b], add=True)
            if with_deg:
                db = j % 2
                if dd[db] is not None:
                    dd[db].wait()
                dd[db] = pltpu.async_copy(onesv, dacc.at[dstv.at[j]],
                                          dsem[db], add=True)
        for desc in (sd + dd) if with_deg else sd:
            if desc is not None:
                desc.wait()
        plsc.subcore_barrier()

        @pl.when(cid == 0)
        def _():
            pltpu.sync_copy(acc.at[pl.ds(base, ROWS_PER_TILE)],
                            out0.at[pl.ds(base, ROWS_PER_TILE)])
            if with_deg:
                pltpu.sync_copy(dacc.at[pl.ds(base, ROWS_PER_TILE)],
                                dg0.at[pl.ds(base, ROWS_PER_TILE)])

        @pl.when(cid == 1)
        def _():
            pltpu.sync_copy(acc.at[pl.ds(base, ROWS_PER_TILE)],
                            out1.at[pl.ds(base, ROWS_PER_TILE)])
            if with_deg:
                pltpu.sync_copy(dacc.at[pl.ds(base, ROWS_PER_TILE)],
                                dg1.at[pl.ds(base, ROWS_PER_TILE)])

    return pl.kernel(body, out_type=out_type, mesh=mesh,
                     scratch_types=scratch,
                     compiler_params=pltpu.CompilerParams(
                         use_tc_tiling_on_sc=tc_tiling))


_PAD_BLOCKS = None


def _pad_blocks():
    global _PAD_BLOCKS
    if _PAD_BLOCKS is None:
        import numpy as np
        n = (NBLK_PAD - NBLK) * EB
        ar = np.arange(n, dtype=np.int32)
        _PAD_BLOCKS = jnp.asarray(np.stack([
            (ar * 7919) % N_NODES,
            N_NODES + ar % (N_PAD - N_NODES),
        ]).reshape(2, NBLK_PAD - NBLK, EB))
    return _PAD_BLOCKS


def kernel(x, edge_index, W1_l, b1, W1_r, W2_l, b2, W2_r):
    eidx = jnp.concatenate(
        [edge_index.astype(jnp.int32).reshape(2, NBLK, EB), _pad_blocks()],
        axis=1)

    p1, q1 = _dense1(x, W1_l, W1_r, b1.reshape(1, HID_DIM))
    s0, s1, dg0, dg1 = _make_sc_scatter(HID_DIM, True, False, 6)(p1, eidx)
    p2, q2, invd = _mid(s0, s1, dg0.reshape(N_PAD, 1), dg1.reshape(N_PAD, 1),
                        q1, W2_l, W2_r, b2.reshape(1, OUT_DIM))
    t0, t1 = _make_sc_scatter(OUT_DIM, False, False, 6)(p2, eidx)
    (out,) = _final(t0, t1, invd, q2)
    return out
